# Initial kernel scaffold; baseline (speedup 1.0000x reference)
import jax
import jax.numpy as jnp
from jax import lax
from jax.experimental import pallas as pl
from jax.experimental.pallas import tpu as pltpu
from jax.experimental.pallas import tpu_sc as plsc

N = 10000
D = 128
NB = 4
NL = 2
NC = 2
NS = 16
NW = NC * NS
NP = 10240
ROWS_PT = NP // NS
C = 128


def _leaky(v):
    return jnp.where(v >= 0.0, v, 0.2 * v)


def _make_mm(xb):
    def body(x_ref, w_ref, as_ref, ad_ref, h_ref, asrc_ref, adst_ref, m_ref):
        x = x_ref[0]
        w = w_ref[0]
        h = jnp.dot(x, w, preferred_element_type=jnp.float32)
        h_ref[0] = h
        a_s = jnp.sum(h * as_ref[0], axis=1)
        a_d = jnp.sum(h * ad_ref[0], axis=1)
        asrc_ref[0, 0] = a_s
        adst_ref[0, 0] = a_d
        mm = _leaky(jnp.max(a_s) + jnp.max(a_d))
        m_ref[0, 0] = jnp.broadcast_to(mm, (D,))

    return pl.pallas_call(
        body,
        grid=(NB,),
        in_specs=[
            pl.BlockSpec((1, NP, D), lambda b: (b if xb > 1 else 0, 0, 0)),
            pl.BlockSpec((1, D, D), lambda b: (b, 0, 0)),
            pl.BlockSpec((1, 1, D), lambda b: (b, 0, 0)),
            pl.BlockSpec((1, 1, D), lambda b: (b, 0, 0)),
        ],
        out_specs=[
            pl.BlockSpec((1, NP, D), lambda b: (b, 0, 0)),
            pl.BlockSpec((1, 1, NP), lambda b: (b, 0, 0)),
            pl.BlockSpec((1, 1, NP), lambda b: (b, 0, 0)),
            pl.BlockSpec((1, 1, D), lambda b: (b, 0, 0)),
        ],
        out_shape=[
            jax.ShapeDtypeStruct((NB, NP, D), jnp.float32),
            jax.ShapeDtypeStruct((NB, 1, NP), jnp.float32),
            jax.ShapeDtypeStruct((NB, 1, NP), jnp.float32),
            jax.ShapeDtypeStruct((NB, 1, D), jnp.float32),
        ],
    )


def _make_post():
    def body(acc_ref, den_ref, bias_ref, pa_ref, xo_ref):
        acc = acc_ref[0, 0] + acc_ref[1, 0]
        den = jnp.sum(den_ref[0], axis=(0, 1))
        den = jnp.maximum(den, 1e-30)
        y = acc / den[:, None] + bias_ref[0, 0]
        pa = pa_ref[0]
        xo_ref[0] = jnp.where(y >= 0.0, y, pa * y)

    return pl.pallas_call(
        body,
        grid=(NB,),
        in_specs=[
            pl.BlockSpec((NC, 1, NP, D), lambda b: (0, b, 0, 0)),
            pl.BlockSpec((1, NC, NS, NP), lambda b: (b, 0, 0, 0)),
            pl.BlockSpec((1, 1, D), lambda b: (b, 0, 0)),
            pl.BlockSpec((1, D), lambda b: (0, 0)),
        ],
        out_specs=[pl.BlockSpec((1, NP, D), lambda b: (b, 0, 0))],
        out_shape=[jax.ShapeDtypeStruct((NB, NP, D), jnp.float32)],
    )


def _make_sc(ncw, ept):
    def body(h2_hbm, asrc_hbm, adst_hbm, m_hbm, src_hbm, dst_hbm, zrows_hbm,
             zn_hbm, out_hbm, den_hbm, asrc_v, adst_v, m_v, denp_v, src_v,
             srcb_v, dst_v, p_v, rows_v, acc_s, sem):
        c = lax.axis_index("c")
        s = lax.axis_index("s")
        t = s * NC + c
        base = t * ept

        for b in range(NB):
            pltpu.sync_copy(asrc_hbm.at[b], asrc_v)
            pltpu.sync_copy(adst_hbm.at[b], adst_v)
            pltpu.sync_copy(m_hbm.at[b, pl.ds(0, 16)], m_v)
            pltpu.sync_copy(zn_hbm, denp_v)
            pltpu.sync_copy(zrows_hbm, acc_s.at[pl.ds(s * ROWS_PT, ROWS_PT)])
            plsc.subcore_barrier()

            def chunk(k, carry):
                off = base + k * C
                pltpu.sync_copy(src_hbm.at[b, pl.ds(off, C)], src_v)
                pltpu.sync_copy(dst_hbm.at[b, pl.ds(off, C)], dst_v)
                mv = m_v[...]
                for j in range(C // 16):
                    sl = pl.ds(j * 16, 16)
                    si = src_v[sl]
                    di = dst_v[sl]
                    av = plsc.load_gather(asrc_v, [si])
                    dv = plsc.load_gather(adst_v, [di])
                    p = jnp.exp(_leaky(av + dv) - mv)
                    p_v[sl] = p
                    plsc.addupdate_scatter(denp_v, [di], p)
                    srcb_v[sl] = si + (b * NP)
                pltpu.async_copy(h2_hbm.at[srcb_v], rows_v, sem).wait()

                def srow(r, rc):
                    pb = plsc.load_gather(p_v, [jnp.full((16,), r, jnp.int32)])
                    for f in range(D // 16):
                        fl = pl.ds(f * 16, 16)
                        rows_v[r, fl] = rows_v[r, fl] * pb
                    return rc

                lax.fori_loop(0, C, srow, 0)
                pltpu.sync_copy(rows_v, acc_s.at[dst_v], add=True)
                return carry

            lax.fori_loop(0, ncw, chunk, 0)

            didx = (b * NC + c) * NS + s
            pltpu.sync_copy(denp_v, den_hbm.at[pl.ds(didx * NP, NP)])
            plsc.subcore_barrier()
            ridx = (c * NB + b) * NP + s * ROWS_PT
            pltpu.sync_copy(acc_s.at[pl.ds(s * ROWS_PT, ROWS_PT)],
                            out_hbm.at[pl.ds(ridx, ROWS_PT)])
            plsc.subcore_barrier()

    return pl.kernel(
        body,
        out_type=[
            jax.ShapeDtypeStruct((NC * NB * NP, D), jnp.float32),
            jax.ShapeDtypeStruct((NB * NC * NS * NP,), jnp.float32),
        ],
        mesh=plsc.VectorSubcoreMesh(core_axis_name="c", subcore_axis_name="s"),
        scratch_types=[
            pltpu.VMEM((NP,), jnp.float32),
            pltpu.VMEM((NP,), jnp.float32),
            pltpu.VMEM((16,), jnp.float32),
            pltpu.VMEM((NP,), jnp.float32),
            pltpu.VMEM((C,), jnp.int32),
            pltpu.VMEM((C,), jnp.int32),
            pltpu.VMEM((C,), jnp.int32),
            pltpu.VMEM((C,), jnp.float32),
            pltpu.VMEM((C, D), jnp.float32),
            pltpu.VMEM_SHARED((NP, D), jnp.float32),
            pltpu.SemaphoreType.DMA,
        ],
    )


def kernel(x, edge_index_g1_pos, edge_index_g2_pos, edge_index_g1_neg,
           edge_index_g2_neg, W_pos, att_src_pos, att_dst_pos, b_pos, W_neg,
           att_src_neg, att_dst_neg, b_neg, prelu_a):
    e = edge_index_g1_pos.shape[1]
    esl = e + N
    ep = -((-esl) // (NW * C)) * (NW * C)
    ept = ep // NW
    ncw = ept // C

    xp = jnp.pad(x, ((0, NP - N), (0, 0)))
    loops = jnp.arange(N, dtype=jnp.int32)
    padi = jnp.full((ep - esl,), N, dtype=jnp.int32)
    srcs, dsts = [], []
    for ei in (edge_index_g1_pos, edge_index_g2_pos, edge_index_g1_neg,
               edge_index_g2_neg):
        srcs.append(jnp.concatenate([ei[0], loops, padi]))
        dsts.append(jnp.concatenate([ei[1], loops, padi]))
    src_all = jnp.stack(srcs)
    dst_all = jnp.stack(dsts)

    w_l = [jnp.stack([W_pos[l], W_pos[l], W_neg[l], W_neg[l]])
           for l in range(NL)]
    as_l = [jnp.stack([att_src_pos[l], att_src_pos[l], att_src_neg[l],
                       att_src_neg[l]]).reshape(NB, 1, D) for l in range(NL)]
    ad_l = [jnp.stack([att_dst_pos[l], att_dst_pos[l], att_dst_neg[l],
                       att_dst_neg[l]]).reshape(NB, 1, D) for l in range(NL)]
    bias_l = [jnp.stack([b_pos[l], b_pos[l], b_neg[l], b_neg[l]]
                        ).reshape(NB, 1, D) for l in range(NL)]
    pa_row = jnp.broadcast_to(prelu_a.astype(jnp.float32), (1, D))
    zrows = jnp.zeros((ROWS_PT, D), jnp.float32)
    zn = jnp.zeros((NP,), jnp.float32)

    sc_call = _make_sc(ncw, ept)
    post_call = _make_post()

    xc = xp[None]
    for l in range(NL):
        h, asrc, adst, m = _make_mm(xc.shape[0])(xc, w_l[l], as_l[l], ad_l[l])
        out_flat, den_flat = sc_call(
            h.reshape(NB * NP, D), asrc.reshape(NB, NP), adst.reshape(NB, NP),
            m.reshape(NB, D), src_all, dst_all, zrows, zn)
        (xc,) = post_call(out_flat.reshape(NC, NB, NP, D),
                          den_flat.reshape(NB, NC, NS, NP), bias_l[l], pa_row)
    return (xc[0, :N], xc[1, :N], xc[2, :N], xc[3, :N])


# speedup vs baseline: 19.6781x; 19.6781x over previous
import jax
import jax.numpy as jnp
from jax import lax
from jax.experimental import pallas as pl
from jax.experimental.pallas import tpu as pltpu
from jax.experimental.pallas import tpu_sc as plsc

N = 10000
D = 128
NB = 4
NL = 2
NC = 2
NS = 16
NW = NC * NS
NP = 10240
ROWS_PT = NP // NS
C = 128


def _leaky(v):
    return jnp.where(v >= 0.0, v, 0.2 * v)


def _make_mm(xb):
    def body(x_ref, w_ref, as_ref, ad_ref, h_ref, asrc_ref, adst_ref, m_ref):
        x = x_ref[0]
        w = w_ref[0]
        h = jnp.dot(x, w, preferred_element_type=jnp.float32)
        h_ref[0] = h
        a_s = jnp.sum(h * as_ref[0], axis=1)
        a_d = jnp.sum(h * ad_ref[0], axis=1)
        asrc_ref[0, 0] = a_s
        adst_ref[0, 0] = a_d
        mm = _leaky(jnp.max(a_s) + jnp.max(a_d))
        m_ref[0, 0] = jnp.broadcast_to(mm, (D,))

    return pl.pallas_call(
        body,
        grid=(NB,),
        in_specs=[
            pl.BlockSpec((1, NP, D), lambda b: (b if xb > 1 else 0, 0, 0)),
            pl.BlockSpec((1, D, D), lambda b: (b, 0, 0)),
            pl.BlockSpec((1, 1, D), lambda b: (b, 0, 0)),
            pl.BlockSpec((1, 1, D), lambda b: (b, 0, 0)),
        ],
        out_specs=[
            pl.BlockSpec((1, NP, D), lambda b: (b, 0, 0)),
            pl.BlockSpec((1, 1, NP), lambda b: (b, 0, 0)),
            pl.BlockSpec((1, 1, NP), lambda b: (b, 0, 0)),
            pl.BlockSpec((1, 1, D), lambda b: (b, 0, 0)),
        ],
        out_shape=[
            jax.ShapeDtypeStruct((NB, NP, D), jnp.float32),
            jax.ShapeDtypeStruct((NB, 1, NP), jnp.float32),
            jax.ShapeDtypeStruct((NB, 1, NP), jnp.float32),
            jax.ShapeDtypeStruct((NB, 1, D), jnp.float32),
        ],
    )


def _make_post():
    def body(acc_ref, den_ref, bias_ref, pa_ref, xo_ref):
        acc = acc_ref[0, 0] + acc_ref[1, 0]
        den = jnp.sum(den_ref[0], axis=(0, 1))
        den = jnp.maximum(den, 1e-30)
        y = acc / den[:, None] + bias_ref[0, 0]
        pa = pa_ref[0]
        xo_ref[0] = jnp.where(y >= 0.0, y, pa * y)

    return pl.pallas_call(
        body,
        grid=(NB,),
        in_specs=[
            pl.BlockSpec((NC, 1, NP, D), lambda b: (0, b, 0, 0)),
            pl.BlockSpec((1, NC, NS, NP), lambda b: (b, 0, 0, 0)),
            pl.BlockSpec((1, 1, D), lambda b: (b, 0, 0)),
            pl.BlockSpec((1, D), lambda b: (0, 0)),
        ],
        out_specs=[pl.BlockSpec((1, NP, D), lambda b: (b, 0, 0))],
        out_shape=[jax.ShapeDtypeStruct((NB, NP, D), jnp.float32)],
    )


def _make_sc(ncw, ept):
    def body(h2_hbm, asrc_hbm, adst_hbm, m_hbm, src_hbm, dst_hbm, zrows_hbm,
             zn_hbm, out_hbm, den_hbm, asrc_v, adst_v, m_v, denp_v, src_v,
             srcb_v, dst_v, p_v, rows_v, acc_s, sem):
        c = lax.axis_index("c")
        s = lax.axis_index("s")
        t = s * NC + c
        base = t * ept

        for b in range(NB):
            pltpu.sync_copy(asrc_hbm.at[b], asrc_v)
            pltpu.sync_copy(adst_hbm.at[b], adst_v)
            pltpu.sync_copy(m_hbm.at[b, pl.ds(0, 16)], m_v)
            pltpu.sync_copy(zn_hbm, denp_v)
            pltpu.sync_copy(zrows_hbm, acc_s.at[pl.ds(s * ROWS_PT, ROWS_PT)])
            plsc.subcore_barrier()

            def chunk(k, carry):
                off = base + k * C
                pltpu.sync_copy(src_hbm.at[b, pl.ds(off, C)], src_v)
                pltpu.sync_copy(dst_hbm.at[b, pl.ds(off, C)], dst_v)
                mv = m_v[...]
                for j in range(C // 16):
                    sl = pl.ds(j * 16, 16)
                    si = src_v[sl]
                    di = dst_v[sl]
                    av = plsc.load_gather(asrc_v, [si])
                    dv = plsc.load_gather(adst_v, [di])
                    p = jnp.exp(_leaky(av + dv) - mv)
                    p_v[sl] = p
                    plsc.addupdate_scatter(denp_v, [di], p)
                    srcb_v[sl] = si + (b * NP)
                pltpu.async_copy(h2_hbm.at[srcb_v], rows_v, sem).wait()

                def srow(r, rc):
                    pb = plsc.load_gather(p_v, [jnp.full((16,), r, jnp.int32)])
                    for f in range(D // 16):
                        fl = pl.ds(f * 16, 16)
                        rows_v[r, fl] = rows_v[r, fl] * pb
                    return rc

                lax.fori_loop(0, C, srow, 0)
                pltpu.sync_copy(rows_v, acc_s.at[dst_v], add=True)
                return carry

            lax.fori_loop(0, ncw, chunk, 0)

            didx = (b * NC + c) * NS + s
            pltpu.sync_copy(denp_v, den_hbm.at[pl.ds(didx * NP, NP)])
            plsc.subcore_barrier()
            ridx = (c * NB + b) * NP + s * ROWS_PT
            pltpu.sync_copy(acc_s.at[pl.ds(s * ROWS_PT, ROWS_PT)],
                            out_hbm.at[pl.ds(ridx, ROWS_PT)])
            plsc.subcore_barrier()

    return pl.kernel(
        body,
        out_type=[
            jax.ShapeDtypeStruct((NC * NB * NP, D), jnp.float32),
            jax.ShapeDtypeStruct((NB * NC * NS * NP,), jnp.float32),
        ],
        mesh=plsc.VectorSubcoreMesh(core_axis_name="c", subcore_axis_name="s"),
        compiler_params=pltpu.CompilerParams(needs_layout_passes=False),
        scratch_types=[
            pltpu.VMEM((NP,), jnp.float32),
            pltpu.VMEM((NP,), jnp.float32),
            pltpu.VMEM((16,), jnp.float32),
            pltpu.VMEM((NP,), jnp.float32),
            pltpu.VMEM((C,), jnp.int32),
            pltpu.VMEM((C,), jnp.int32),
            pltpu.VMEM((C,), jnp.int32),
            pltpu.VMEM((C,), jnp.float32),
            pltpu.VMEM((C, D), jnp.float32),
            pltpu.VMEM_SHARED((NP, D), jnp.float32),
            pltpu.SemaphoreType.DMA,
        ],
    )


def kernel(x, edge_index_g1_pos, edge_index_g2_pos, edge_index_g1_neg,
           edge_index_g2_neg, W_pos, att_src_pos, att_dst_pos, b_pos, W_neg,
           att_src_neg, att_dst_neg, b_neg, prelu_a):
    e = edge_index_g1_pos.shape[1]
    esl = e + N
    ep = -((-esl) // (NW * C)) * (NW * C)
    ept = ep // NW
    ncw = ept // C

    xp = jnp.pad(x, ((0, NP - N), (0, 0)))
    loops = jnp.arange(N, dtype=jnp.int32)
    padi = jnp.full((ep - esl,), N, dtype=jnp.int32)
    srcs, dsts = [], []
    for ei in (edge_index_g1_pos, edge_index_g2_pos, edge_index_g1_neg,
               edge_index_g2_neg):
        srcs.append(jnp.concatenate([ei[0], loops, padi]))
        dsts.append(jnp.concatenate([ei[1], loops, padi]))
    src_all = jnp.stack(srcs)
    dst_all = jnp.stack(dsts)

    w_l = [jnp.stack([W_pos[l], W_pos[l], W_neg[l], W_neg[l]])
           for l in range(NL)]
    as_l = [jnp.stack([att_src_pos[l], att_src_pos[l], att_src_neg[l],
                       att_src_neg[l]]).reshape(NB, 1, D) for l in range(NL)]
    ad_l = [jnp.stack([att_dst_pos[l], att_dst_pos[l], att_dst_neg[l],
                       att_dst_neg[l]]).reshape(NB, 1, D) for l in range(NL)]
    bias_l = [jnp.stack([b_pos[l], b_pos[l], b_neg[l], b_neg[l]]
                        ).reshape(NB, 1, D) for l in range(NL)]
    pa_row = jnp.broadcast_to(prelu_a.astype(jnp.float32), (1, D))
    zrows = jnp.zeros((ROWS_PT, D), jnp.float32)
    zn = jnp.zeros((NP,), jnp.float32)

    sc_call = _make_sc(ncw, ept)
    post_call = _make_post()

    xc = xp[None]
    for l in range(NL):
        h, asrc, adst, m = _make_mm(xc.shape[0])(xc, w_l[l], as_l[l], ad_l[l])
        out_flat, den_flat = sc_call(
            h.reshape(NB * NP, D), asrc.reshape(NB, NP), adst.reshape(NB, NP),
            m.reshape(NB, D), src_all, dst_all, zrows, zn)
        (xc,) = post_call(out_flat.reshape(NC, NB, NP, D),
                          den_flat.reshape(NB, NC, NS, NP), bias_l[l], pa_row)
    return (xc[0, :N], xc[1, :N], xc[2, :N], xc[3, :N])
